# Initial kernel scaffold; baseline (speedup 1.0000x reference)
import jax
import jax.numpy as jnp
from jax.experimental import pallas as pl


def kernel(x_in, sigma, alpha, f_w, f_b, p_w, p_b, mlp_w1, mlp_b1, mlp_w2, mlp_b2):
    raise NotImplementedError("write your pallas kernel here")


# speedup vs baseline: 6.7826x; 6.7826x over previous
import jax
import jax.numpy as jnp
from jax import lax
from jax.experimental import pallas as pl
from jax.experimental.pallas import tpu as pltpu

_WS = 7
_K = 9


def _sigmoid(x):
    return 1.0 / (1.0 + jnp.exp(-x))


def _patch_body(scal_ref, x_ref, fw_ref, fb_ref, pw_ref, pb_ref, out_ref,
                *, n, hp, wp, d4):
    x = x_ref[0]
    fw = fw_ref[...]
    f = jnp.dot(fw, x, preferred_element_type=jnp.float32) + fb_ref[...]

    nrm = jnp.sqrt(jnp.sum(f * f, axis=0, keepdims=True))
    nrm = jnp.maximum(nrm, 1e-8)
    xn = f / nrm
    sim = lax.dot_general(xn, xn, (((0,), (0,)), ((), ())),
                          preferred_element_type=jnp.float32)

    sigma = scal_ref[0]
    alpha = scal_ref[1]

    rows = lax.broadcasted_iota(jnp.int32, (n, n), 0)
    cols = lax.broadcasted_iota(jnp.int32, (n, n), 1)
    ri = rows // wp
    ci = rows % wp
    rj = cols // wp
    cj = cols % wp
    d2 = ((ri - rj) * (ri - rj) + (ci - cj) * (ci - cj)).astype(jnp.float32)
    dist = jnp.sqrt(d2)
    sdist = jnp.exp(-(dist * dist) / (2.0 * sigma * sigma))

    work = alpha * sim + (1.0 - alpha) * sdist

    w1 = [[scal_ref[2 + 2 * c], scal_ref[3 + 2 * c]] for c in range(4)]
    b1 = [scal_ref[10 + c] for c in range(4)]
    w2 = [scal_ref[14 + c] for c in range(4)]
    b2 = scal_ref[18]

    i_col = lax.broadcasted_iota(jnp.int32, (n, 1), 0)
    ri1 = i_col // wp
    ci1 = i_col % wp

    wd = jnp.zeros((n, n), jnp.float32)
    wsum = jnp.zeros((n, 1), jnp.float32)

    for _ in range(_K):
        m = jnp.max(work, axis=1, keepdims=True)
        ism = work == m
        jsel = jnp.min(jnp.where(ism, cols, n), axis=1, keepdims=True)
        first = cols == jsel
        rj1 = jsel // wp
        cj1 = jsel % wp
        e2 = ((ri1 - rj1) * (ri1 - rj1) + (ci1 - cj1) * (ci1 - cj1)).astype(jnp.float32)
        ed = jnp.sqrt(e2)
        sd_t = jnp.exp(-(ed * ed) / (2.0 * sigma * sigma))
        sf_t = (m - (1.0 - alpha) * sd_t) / alpha
        u = b2
        for c in range(4):
            h = sf_t * w1[c][0] + sd_t * w1[c][1] + b1[c]
            h = h * _sigmoid(h)
            u = u + h * w2[c]
        w_t = _sigmoid(u)
        wd = jnp.where(first, w_t, wd)
        wsum = wsum + w_t
        work = jnp.where(first, -jnp.inf, work)

    wdn = wd * (1.0 / (wsum + 1e-12))
    od = lax.dot_general(wdn, f, (((1,), (1,)), ((), ())),
                         preferred_element_type=jnp.float32)
    oc = lax.dot_general(pw_ref[...], od, (((1,), (1,)), ((), ())),
                         preferred_element_type=jnp.float32)
    out_ref[0] = oc + pb_ref[...]


def kernel(x_in, sigma, alpha, f_w, f_b, p_w, p_b, mlp_w1, mlp_b1, mlp_w2, mlp_b2):
    B, C, H, W = x_in.shape
    ws = _WS
    hp, wp = H // ws, W // ws
    n = hp * wp
    bp = B * ws * ws
    d4 = f_w.shape[0]

    xp = x_in.reshape(C, ws, hp, ws, wp).transpose(1, 3, 0, 2, 4).reshape(bp, C, n)
    scal = jnp.concatenate([
        sigma.reshape(1), alpha.reshape(1),
        mlp_w1.reshape(-1), mlp_b1.reshape(-1),
        mlp_w2.reshape(-1), mlp_b2.reshape(-1),
    ]).astype(jnp.float32)

    import functools
    body = functools.partial(_patch_body, n=n, hp=hp, wp=wp, d4=d4)

    y = pl.pallas_call(
        body,
        grid=(bp,),
        in_specs=[
            pl.BlockSpec(memory_space=pltpu.SMEM),
            pl.BlockSpec((1, C, n), lambda i: (i, 0, 0)),
            pl.BlockSpec((d4, C), lambda i: (0, 0)),
            pl.BlockSpec((d4, 1), lambda i: (0, 0)),
            pl.BlockSpec((C, d4), lambda i: (0, 0)),
            pl.BlockSpec((C, 1), lambda i: (0, 0)),
        ],
        out_specs=pl.BlockSpec((1, C, n), lambda i: (i, 0, 0)),
        out_shape=jax.ShapeDtypeStruct((bp, C, n), jnp.float32),
    )(scal, xp, f_w, f_b.reshape(d4, 1), p_w, p_b.reshape(C, 1))

    out = y.reshape(ws, ws, C, hp, wp).transpose(2, 0, 3, 1, 4).reshape(B, C, H * W)
    return out


# speedup vs baseline: 15.7659x; 2.3245x over previous
import functools

import jax
import jax.numpy as jnp
from jax import lax
from jax.experimental import pallas as pl
from jax.experimental.pallas import tpu as pltpu

_WS = 7
_K = 9


def _sigmoid(x):
    return 1.0 / (1.0 + jnp.exp(-x))


def _patch_body(scal_ref, x_ref, fw_ref, fb_ref, pw_ref, pb_ref, out_ref,
                *, n, hp, wp, d4):
    x = x_ref[0]
    fw = fw_ref[...]
    f = jnp.dot(fw, x, preferred_element_type=jnp.float32) + fb_ref[...]

    nrm = jnp.sqrt(jnp.sum(f * f, axis=0, keepdims=True))
    nrm = jnp.maximum(nrm, 1e-8)
    xn = f / nrm
    sim = lax.dot_general(xn, xn, (((0,), (0,)), ((), ())),
                          preferred_element_type=jnp.float32)

    sigma = scal_ref[0]
    alpha = scal_ref[1]

    rows = lax.broadcasted_iota(jnp.int32, (n, n), 0)
    cols = lax.broadcasted_iota(jnp.int32, (n, n), 1)
    ri = rows // wp
    ci = rows % wp
    rj = cols // wp
    cj = cols % wp
    d2 = ((ri - rj) * (ri - rj) + (ci - cj) * (ci - cj)).astype(jnp.float32)
    dist = jnp.sqrt(d2)
    sdist = jnp.exp(-(dist * dist) / (2.0 * sigma * sigma))

    work = alpha * sim + (1.0 - alpha) * sdist

    w1 = [[scal_ref[2 + 2 * c], scal_ref[3 + 2 * c]] for c in range(4)]
    b1 = [scal_ref[10 + c] for c in range(4)]
    w2 = [scal_ref[14 + c] for c in range(4)]
    b2 = scal_ref[18]

    i_row = lax.broadcasted_iota(jnp.int32, (1, n), 1)
    ri1 = i_row // wp
    ci1 = i_row % wp

    wd = jnp.zeros((n, n), jnp.float32)
    wsum = jnp.zeros((1, n), jnp.float32)

    for _ in range(_K):
        m = jnp.max(work, axis=0, keepdims=True)
        ism = work == m
        jsel = jnp.min(jnp.where(ism, rows, n), axis=0, keepdims=True)
        first = rows == jsel
        rj1 = jsel // wp
        cj1 = jsel % wp
        e2 = ((ri1 - rj1) * (ri1 - rj1) + (ci1 - cj1) * (ci1 - cj1)).astype(jnp.float32)
        ed = jnp.sqrt(e2)
        sd_t = jnp.exp(-(ed * ed) / (2.0 * sigma * sigma))
        sf_t = (m - (1.0 - alpha) * sd_t) / alpha
        u = b2
        for c in range(4):
            h = sf_t * w1[c][0] + sd_t * w1[c][1] + b1[c]
            h = h * _sigmoid(h)
            u = u + h * w2[c]
        w_t = _sigmoid(u)
        wd = jnp.where(first, w_t, wd)
        wsum = wsum + w_t
        work = jnp.where(first, -jnp.inf, work)

    wdn = wd * (1.0 / (wsum + 1e-12))
    od = lax.dot_general(f, wdn, (((1,), (0,)), ((), ())),
                         preferred_element_type=jnp.float32)
    oc = jnp.dot(pw_ref[...], od, preferred_element_type=jnp.float32)
    out_ref[0] = oc + pb_ref[...]


def kernel(x_in, sigma, alpha, f_w, f_b, p_w, p_b, mlp_w1, mlp_b1, mlp_w2, mlp_b2):
    B, C, H, W = x_in.shape
    ws = _WS
    hp, wp = H // ws, W // ws
    n = hp * wp
    bp = B * ws * ws
    d4 = f_w.shape[0]

    xp = x_in.reshape(C, ws, hp, ws, wp).transpose(1, 3, 0, 2, 4).reshape(bp, C, n)
    scal = jnp.concatenate([
        sigma.reshape(1), alpha.reshape(1),
        mlp_w1.reshape(-1), mlp_b1.reshape(-1),
        mlp_w2.reshape(-1), mlp_b2.reshape(-1),
    ]).astype(jnp.float32)

    body = functools.partial(_patch_body, n=n, hp=hp, wp=wp, d4=d4)

    y = pl.pallas_call(
        body,
        grid=(bp,),
        in_specs=[
            pl.BlockSpec(memory_space=pltpu.SMEM),
            pl.BlockSpec((1, C, n), lambda i: (i, 0, 0)),
            pl.BlockSpec((d4, C), lambda i: (0, 0)),
            pl.BlockSpec((d4, 1), lambda i: (0, 0)),
            pl.BlockSpec((C, d4), lambda i: (0, 0)),
            pl.BlockSpec((C, 1), lambda i: (0, 0)),
        ],
        out_specs=pl.BlockSpec((1, C, n), lambda i: (i, 0, 0)),
        out_shape=jax.ShapeDtypeStruct((bp, C, n), jnp.float32),
    )(scal, xp, f_w, f_b.reshape(d4, 1), p_w, p_b.reshape(C, 1))

    out = y.reshape(ws, ws, C, hp, wp).transpose(2, 0, 3, 1, 4).reshape(B, C, H * W)
    return out


# speedup vs baseline: 32.2077x; 2.0429x over previous
import functools

import jax
import jax.numpy as jnp
from jax import lax
from jax.experimental import pallas as pl
from jax.experimental.pallas import tpu as pltpu

_WS = 7
_K = 9


def _sigmoid(x):
    return 1.0 / (1.0 + jnp.exp(-x))


def _graph_body(fT, scal_ref, *, n, wp):
    f2 = jnp.transpose(fT, (1, 0))
    nrm = jnp.sqrt(jnp.sum(f2 * f2, axis=0, keepdims=True))
    nrm = jnp.maximum(nrm, 1e-8)
    xn = f2 / nrm
    sim = lax.dot_general(xn, xn, (((0,), (0,)), ((), ())),
                          preferred_element_type=jnp.float32)

    sigma = scal_ref[0]
    alpha = scal_ref[1]

    rows = lax.broadcasted_iota(jnp.int32, (n, n), 0)
    cols = lax.broadcasted_iota(jnp.int32, (n, n), 1)
    ri = rows // wp
    ci = rows % wp
    rj = cols // wp
    cj = cols % wp
    d2 = ((ri - rj) * (ri - rj) + (ci - cj) * (ci - cj)).astype(jnp.float32)
    dist = jnp.sqrt(d2)
    sdist = jnp.exp(-(dist * dist) / (2.0 * sigma * sigma))

    work = alpha * sim + (1.0 - alpha) * sdist

    ki = lax.convert_element_type(jnp.round(work * jnp.float32(2.0**22)),
                                  jnp.int32)
    key = (ki << 8) | (jnp.int32(255) - rows)
    neg = jnp.int32(-2147483648)

    w1 = [[scal_ref[2 + 2 * c_], scal_ref[3 + 2 * c_]] for c_ in range(4)]
    b1 = [scal_ref[10 + c_] for c_ in range(4)]
    w2 = [scal_ref[14 + c_] for c_ in range(4)]
    b2 = scal_ref[18]

    i_row = lax.broadcasted_iota(jnp.int32, (1, n), 1)
    ri1 = i_row // wp
    ci1 = i_row % wp

    wd = jnp.zeros((n, n), jnp.float32)
    wsum = jnp.zeros((1, n), jnp.float32)

    for _ in range(_K):
        kmax = jnp.max(key, axis=0, keepdims=True)
        jsel = jnp.int32(255) - (kmax & jnp.int32(255))
        first = rows == jsel
        key = jnp.where(first, neg, key)
        m = lax.convert_element_type(kmax >> 8, jnp.float32) * jnp.float32(2.0**-22)
        rj1 = jsel // wp
        cj1 = jsel % wp
        e2 = ((ri1 - rj1) * (ri1 - rj1) + (ci1 - cj1) * (ci1 - cj1)).astype(jnp.float32)
        ed = jnp.sqrt(e2)
        sd_t = jnp.exp(-(ed * ed) / (2.0 * sigma * sigma))
        sf_t = (m - (1.0 - alpha) * sd_t) / alpha
        u = b2
        for c_ in range(4):
            h = sf_t * w1[c_][0] + sd_t * w1[c_][1] + b1[c_]
            h = h * _sigmoid(h)
            u = u + h * w2[c_]
        w_t = _sigmoid(u)
        wd = jnp.where(first, w_t, wd)
        wsum = wsum + w_t

    wdn = wd * (1.0 / (wsum + 1e-12))
    return lax.dot_general(wdn, fT, (((0,), (0,)), ((), ())),
                           preferred_element_type=jnp.float32)


def _strip_body(scal_ref, x_ref, fwt_ref, fb_ref, out_ref, *, n, hp, wp, ws):
    d4 = fwt_ref.shape[1]
    fT = lax.dot_general(x_ref[...], fwt_ref[...], (((0,), (0,)), ((), ())),
                         preferred_element_type=jnp.float32) + fb_ref[...]
    f4 = fT.reshape(hp, ws, wp, d4)
    pieces = []
    for hg in range(ws):
        fTp = f4[:, hg].reshape(n, d4)
        odTp = _graph_body(fTp, scal_ref, n=n, wp=wp)
        pieces.append(odTp.reshape(hp, 1, wp, d4))
    out_ref[0] = jnp.concatenate(pieces, axis=1)


def _proj_body(od_ref, pw_ref, pb_ref, out_ref):
    odc = jnp.transpose(od_ref[...], (1, 0))
    oc = jnp.dot(pw_ref[...], odc, preferred_element_type=jnp.float32)
    out_ref[...] = oc + pb_ref[...]


def kernel(x_in, sigma, alpha, f_w, f_b, p_w, p_b, mlp_w1, mlp_b1, mlp_w2, mlp_b2):
    B, C, H, W = x_in.shape
    ws = _WS
    hp, wp = H // ws, W // ws
    n = hp * wp
    d4 = f_w.shape[0]
    strip = hp * W

    X = x_in.reshape(C, H * W)
    fw_t = f_w.T
    scal = jnp.concatenate([
        sigma.reshape(1), alpha.reshape(1),
        mlp_w1.reshape(-1), mlp_b1.reshape(-1),
        mlp_w2.reshape(-1), mlp_b2.reshape(-1),
    ]).astype(jnp.float32)

    sbody = functools.partial(_strip_body, n=n, hp=hp, wp=wp, ws=ws)
    od = pl.pallas_call(
        sbody,
        grid=(ws,),
        in_specs=[
            pl.BlockSpec(memory_space=pltpu.SMEM),
            pl.BlockSpec((C, strip), lambda i: (0, i)),
            pl.BlockSpec((C, d4), lambda i: (0, 0)),
            pl.BlockSpec((1, d4), lambda i: (0, 0)),
        ],
        out_specs=pl.BlockSpec((1, hp, ws, wp, d4), lambda i: (i, 0, 0, 0, 0)),
        out_shape=jax.ShapeDtypeStruct((ws, hp, ws, wp, d4), jnp.float32),
    )(scal, X, fw_t, f_b.reshape(1, d4))

    od2 = od.reshape(H * W, d4)
    y = pl.pallas_call(
        _proj_body,
        grid=(ws,),
        in_specs=[
            pl.BlockSpec((strip, d4), lambda i: (i, 0)),
            pl.BlockSpec((C, d4), lambda i: (0, 0)),
            pl.BlockSpec((C, 1), lambda i: (0, 0)),
        ],
        out_specs=pl.BlockSpec((C, strip), lambda i: (0, i)),
        out_shape=jax.ShapeDtypeStruct((C, H * W), jnp.float32),
    )(od2, p_w, p_b.reshape(C, 1))

    return y.reshape(B, C, H * W)


# speedup vs baseline: 33.4711x; 1.0392x over previous
import functools

import jax
import jax.numpy as jnp
from jax import lax
from jax.experimental import pallas as pl
from jax.experimental.pallas import tpu as pltpu

_WS = 7
_K = 9


def _sigmoid(x):
    return 1.0 / (1.0 + jnp.exp(-x))


def _graph_body(fT, consts, *, n):
    f2 = jnp.transpose(fT, (1, 0))
    nrm = jnp.sqrt(jnp.sum(f2 * f2, axis=0, keepdims=True))
    nrm = jnp.maximum(nrm, 1e-8)
    xn = f2 / nrm
    sim = lax.dot_general(xn, xn, (((0,), (0,)), ((), ())),
                          preferred_element_type=jnp.float32)

    (alpha22, sdist_bias, rows, mlp, sigma, alpha) = consts
    w1, b1, w2, b2 = mlp

    ki = lax.convert_element_type(jnp.round(sim * alpha22 + sdist_bias),
                                  jnp.int32)
    key = (ki << 8) | (jnp.int32(255) - rows)
    neg = jnp.int32(-2147483648)

    wd = jnp.zeros((n, n), jnp.float32)
    wsum = jnp.zeros((1, n), jnp.float32)

    for _ in range(_K):
        kmax = jnp.max(key, axis=0, keepdims=True)
        jsel = jnp.int32(255) - (kmax & jnp.int32(255))
        first = rows == jsel
        key = jnp.where(first, neg, key)
        m = lax.convert_element_type(kmax >> 8, jnp.float32) * jnp.float32(2.0**-22)
        i_row = lax.broadcasted_iota(jnp.int32, (1, n), 1)
        dr = (i_row >> 4) - (jsel >> 4)
        dc = (i_row & 15) - (jsel & 15)
        e2 = (dr * dr + dc * dc).astype(jnp.float32)
        ed = jnp.sqrt(e2)
        sd_t = jnp.exp(-(ed * ed) / (2.0 * sigma * sigma))
        sf_t = (m - (1.0 - alpha) * sd_t) / alpha
        u = b2
        for c_ in range(4):
            h = sf_t * w1[c_][0] + sd_t * w1[c_][1] + b1[c_]
            h = h * _sigmoid(h)
            u = u + h * w2[c_]
        w_t = _sigmoid(u)
        wd = jnp.where(first, w_t, wd)
        wsum = wsum + w_t

    wdn = wd * (1.0 / (wsum + 1e-12))
    return lax.dot_general(wdn, fT, (((0,), (0,)), ((), ())),
                           preferred_element_type=jnp.float32)


def _strip_body(scal_ref, x_ref, fwt_ref, fb_ref, out_ref, *, n, hp, wp, ws):
    d4 = fwt_ref.shape[1]
    fT = lax.dot_general(x_ref[...], fwt_ref[...], (((0,), (0,)), ((), ())),
                         preferred_element_type=jnp.float32) + fb_ref[...]
    f4 = fT.reshape(hp, ws, wp, d4)

    sigma = scal_ref[0]
    alpha = scal_ref[1]
    w1 = [[scal_ref[2 + 2 * c_], scal_ref[3 + 2 * c_]] for c_ in range(4)]
    b1 = [scal_ref[10 + c_] for c_ in range(4)]
    w2 = [scal_ref[14 + c_] for c_ in range(4)]
    b2 = scal_ref[18]

    rows = lax.broadcasted_iota(jnp.int32, (n, n), 0)
    cols = lax.broadcasted_iota(jnp.int32, (n, n), 1)
    dr = (rows >> 4) - (cols >> 4)
    dc = (rows & 15) - (cols & 15)
    d2 = (dr * dr + dc * dc).astype(jnp.float32)
    dist = jnp.sqrt(d2)
    sdist = jnp.exp(-(dist * dist) / (2.0 * sigma * sigma))
    alpha22 = alpha * jnp.float32(2.0**22)
    sdist_bias = (1.0 - alpha) * jnp.float32(2.0**22) * sdist

    consts = (alpha22, sdist_bias, rows, (w1, b1, w2, b2), sigma, alpha)

    pieces = []
    for hg in range(ws):
        fTp = f4[:, hg].reshape(n, d4)
        odTp = _graph_body(fTp, consts, n=n)
        pieces.append(odTp.reshape(hp, 1, wp, d4))
    out_ref[0] = jnp.concatenate(pieces, axis=1)


def _proj_body(od_ref, pw_ref, pb_ref, out_ref):
    odc = jnp.transpose(od_ref[...], (1, 0))
    oc = jnp.dot(pw_ref[...], odc, preferred_element_type=jnp.float32)
    out_ref[...] = oc + pb_ref[...]


def kernel(x_in, sigma, alpha, f_w, f_b, p_w, p_b, mlp_w1, mlp_b1, mlp_w2, mlp_b2):
    B, C, H, W = x_in.shape
    ws = _WS
    hp, wp = H // ws, W // ws
    n = hp * wp
    d4 = f_w.shape[0]
    strip = hp * W

    X = x_in.reshape(C, H * W)
    fw_t = f_w.T
    scal = jnp.concatenate([
        sigma.reshape(1), alpha.reshape(1),
        mlp_w1.reshape(-1), mlp_b1.reshape(-1),
        mlp_w2.reshape(-1), mlp_b2.reshape(-1),
    ]).astype(jnp.float32)

    sbody = functools.partial(_strip_body, n=n, hp=hp, wp=wp, ws=ws)
    od = pl.pallas_call(
        sbody,
        grid=(ws,),
        in_specs=[
            pl.BlockSpec(memory_space=pltpu.SMEM),
            pl.BlockSpec((C, strip), lambda i: (0, i)),
            pl.BlockSpec((C, d4), lambda i: (0, 0)),
            pl.BlockSpec((1, d4), lambda i: (0, 0)),
        ],
        out_specs=pl.BlockSpec((1, hp, ws, wp, d4), lambda i: (i, 0, 0, 0, 0)),
        out_shape=jax.ShapeDtypeStruct((ws, hp, ws, wp, d4), jnp.float32),
    )(scal, X, fw_t, f_b.reshape(1, d4))

    od2 = od.reshape(H * W, d4)
    y = pl.pallas_call(
        _proj_body,
        grid=(ws,),
        in_specs=[
            pl.BlockSpec((strip, d4), lambda i: (i, 0)),
            pl.BlockSpec((C, d4), lambda i: (0, 0)),
            pl.BlockSpec((C, 1), lambda i: (0, 0)),
        ],
        out_specs=pl.BlockSpec((C, strip), lambda i: (0, i)),
        out_shape=jax.ShapeDtypeStruct((C, H * W), jnp.float32),
    )(od2, p_w, p_b.reshape(C, 1))

    return y.reshape(B, C, H * W)


# speedup vs baseline: 36.5286x; 1.0913x over previous
import functools

import jax
import jax.numpy as jnp
from jax import lax
from jax.experimental import pallas as pl
from jax.experimental.pallas import tpu as pltpu

_WS = 7
_K = 9


def _sigmoid(x):
    return 1.0 / (1.0 + jnp.exp(-x))


def _graph_body(fT, consts, *, n):
    f2 = jnp.transpose(fT, (1, 0))
    nrm = jnp.sqrt(jnp.sum(f2 * f2, axis=0, keepdims=True))
    nrm = jnp.maximum(nrm, 1e-8)
    xn = f2 / nrm
    sim = lax.dot_general(xn, xn, (((0,), (0,)), ((), ())),
                          preferred_element_type=jnp.float32)

    (alpha22, sdist_bias, mlp, sigma) = consts
    w1m, w1d, b1, w2, b2 = mlp

    ki = lax.convert_element_type(jnp.round(sim * alpha22 + sdist_bias),
                                  jnp.int32)
    rows = lax.broadcasted_iota(jnp.int32, (n, n), 0)
    keyf = lax.bitcast_convert_type(
        (ki << 8) + (jnp.int32(255) - rows) + jnp.int32(2**30), jnp.float32)

    wd = jnp.zeros((n, n), jnp.float32)
    wsum = jnp.zeros((1, n), jnp.float32)

    for _ in range(_K):
        kmaxf = jnp.max(keyf, axis=0, keepdims=True)
        first = keyf == kmaxf
        keyf = jnp.where(first, jnp.float32(0.0), keyf)
        kmax = lax.bitcast_convert_type(kmaxf, jnp.int32) - jnp.int32(2**30)
        jsel = jnp.int32(255) - (kmax & jnp.int32(255))
        m = lax.convert_element_type(kmax >> 8, jnp.float32) * jnp.float32(2.0**-21)
        i_row = lax.broadcasted_iota(jnp.int32, (1, n), 1)
        dr = (i_row >> 4) - (jsel >> 4)
        dc = (i_row & 15) - (jsel & 15)
        e2 = (dr * dr + dc * dc).astype(jnp.float32)
        sd_t = jnp.exp(-e2 / (2.0 * sigma * sigma))
        u = b2
        for c_ in range(4):
            h = m * w1m[c_] + sd_t * w1d[c_] + b1[c_]
            h = h * _sigmoid(h)
            u = u + h * w2[c_]
        w_t = _sigmoid(u)
        wd = jnp.where(first, w_t, wd)
        wsum = wsum + w_t

    wdn = wd * (1.0 / (wsum + 1e-12))
    return lax.dot_general(wdn, fT, (((0,), (0,)), ((), ())),
                           preferred_element_type=jnp.float32)


def _strip_body(scal_ref, x_ref, fwt_ref, fb_ref, out_ref, *, n, hp, wp, ws):
    d4 = fwt_ref.shape[1]
    fT = lax.dot_general(x_ref[...], fwt_ref[...], (((0,), (0,)), ((), ())),
                         preferred_element_type=jnp.float32) + fb_ref[...]
    f4 = fT.reshape(hp, ws, wp, d4)

    sigma = scal_ref[0]
    alpha = scal_ref[1]
    w1m = [scal_ref[2 + 2 * c_] / alpha for c_ in range(4)]
    w1d = [scal_ref[3 + 2 * c_]
           - scal_ref[2 + 2 * c_] * (1.0 - alpha) / alpha for c_ in range(4)]
    b1 = [scal_ref[10 + c_] for c_ in range(4)]
    w2 = [scal_ref[14 + c_] for c_ in range(4)]
    b2 = scal_ref[18]

    rows = lax.broadcasted_iota(jnp.int32, (n, n), 0)
    cols = lax.broadcasted_iota(jnp.int32, (n, n), 1)
    dr = (rows >> 4) - (cols >> 4)
    dc = (rows & 15) - (cols & 15)
    d2 = (dr * dr + dc * dc).astype(jnp.float32)
    dist = jnp.sqrt(d2)
    sdist = jnp.exp(-(dist * dist) / (2.0 * sigma * sigma))
    alpha22 = alpha * jnp.float32(2.0**21)
    sdist_bias = (1.0 - alpha) * jnp.float32(2.0**21) * sdist

    consts = (alpha22, sdist_bias, (w1m, w1d, b1, w2, b2), sigma)

    pieces = []
    for hg in range(ws):
        fTp = f4[:, hg].reshape(n, d4)
        odTp = _graph_body(fTp, consts, n=n)
        pieces.append(odTp.reshape(hp, 1, wp, d4))
    out_ref[0] = jnp.concatenate(pieces, axis=1)


def _proj_body(od_ref, pw_ref, pb_ref, out_ref):
    odc = jnp.transpose(od_ref[...], (1, 0))
    oc = jnp.dot(pw_ref[...], odc, preferred_element_type=jnp.float32)
    out_ref[...] = oc + pb_ref[...]


def kernel(x_in, sigma, alpha, f_w, f_b, p_w, p_b, mlp_w1, mlp_b1, mlp_w2, mlp_b2):
    B, C, H, W = x_in.shape
    ws = _WS
    hp, wp = H // ws, W // ws
    n = hp * wp
    d4 = f_w.shape[0]
    strip = hp * W

    X = x_in.reshape(C, H * W)
    fw_t = f_w.T
    scal = jnp.concatenate([
        sigma.reshape(1), alpha.reshape(1),
        mlp_w1.reshape(-1), mlp_b1.reshape(-1),
        mlp_w2.reshape(-1), mlp_b2.reshape(-1),
    ]).astype(jnp.float32)

    sbody = functools.partial(_strip_body, n=n, hp=hp, wp=wp, ws=ws)
    od = pl.pallas_call(
        sbody,
        grid=(ws,),
        in_specs=[
            pl.BlockSpec(memory_space=pltpu.SMEM),
            pl.BlockSpec((C, strip), lambda i: (0, i)),
            pl.BlockSpec((C, d4), lambda i: (0, 0)),
            pl.BlockSpec((1, d4), lambda i: (0, 0)),
        ],
        out_specs=pl.BlockSpec((1, hp, ws, wp, d4), lambda i: (i, 0, 0, 0, 0)),
        out_shape=jax.ShapeDtypeStruct((ws, hp, ws, wp, d4), jnp.float32),
    )(scal, X, fw_t, f_b.reshape(1, d4))

    od2 = od.reshape(H * W, d4)
    y = pl.pallas_call(
        _proj_body,
        grid=(ws,),
        in_specs=[
            pl.BlockSpec((strip, d4), lambda i: (i, 0)),
            pl.BlockSpec((C, d4), lambda i: (0, 0)),
            pl.BlockSpec((C, 1), lambda i: (0, 0)),
        ],
        out_specs=pl.BlockSpec((C, strip), lambda i: (0, i)),
        out_shape=jax.ShapeDtypeStruct((C, H * W), jnp.float32),
    )(od2, p_w, p_b.reshape(C, 1))

    return y.reshape(B, C, H * W)
